# Initial kernel scaffold; baseline (speedup 1.0000x reference)
import jax
import jax.numpy as jnp
from jax.experimental import pallas as pl


def kernel(x, y):
    raise NotImplementedError("write your pallas kernel here")


# speedup vs baseline: 639.7477x; 639.7477x over previous
import jax
import jax.numpy as jnp
from jax import lax
from jax.experimental import pallas as pl
from jax.experimental.pallas import tpu as pltpu

_K = 7
_D = 16
_DO = _D - _K + 1
_H = 64
_HO = _H - _K + 1
_C = 3


def _loss_kernel(x_ref, y_ref, out_ref):
    xs = x_ref[...]
    ys = y_ref[...]

    Dm = []
    for delta in range(-(_DO - 1), _DO):
        a_lo = max(0, -delta)
        a_hi = min(_D - 1, _D - 1 - delta)
        na = a_hi - a_lo + 1
        diff = xs[:, a_lo:a_hi + 1] - ys[:, a_lo + delta:a_hi + 1 + delta]
        z = jnp.sum(diff * diff, axis=0)
        zh = z[:, 0:_HO, :]
        for t in range(1, _K):
            zh = zh + z[:, t:t + _HO, :]
        zw = zh[:, :, 0:_HO]
        for t in range(1, _K):
            zw = zw + zh[:, :, t:t + _HO]
        ndi = na - _K + 1
        dsum = zw[0:ndi]
        for t in range(1, _K):
            dsum = dsum + zw[t:t + ndi]
        pad_lo = a_lo
        pad_hi = _DO - ndi - pad_lo
        dsum = jnp.pad(dsum, ((pad_lo, pad_hi), (0, 0), (0, 0)))
        Dm.append(dsum)

    def d_for_dj(dj):
        return jnp.concatenate(
            [Dm[dj - di + _DO - 1][di:di + 1] for di in range(_DO)], axis=0)

    minval = d_for_dj(0)
    minidx = jnp.zeros((_DO, _HO, _HO), dtype=jnp.int32)
    for dj in range(1, _DO):
        dv = d_for_dj(dj)
        pred = dv < minval
        minidx = jnp.where(pred, jnp.int32(dj), minidx)
        minval = jnp.where(pred, dv, minval)

    eye = lax.broadcasted_iota(jnp.int32, (1, _DO, 1, 1), 1)
    I = (minidx[:, None] == eye).astype(jnp.float32)
    Ip = jnp.pad(I, ((0, 0), (0, 0), (6, 6), (6, 6)))
    Jh = Ip[:, :, 0:_H, 6:6 + _HO]
    for t in range(1, _K):
        Jh = Jh + Ip[:, :, t:t + _H, 6:6 + _HO]
    Jp = jnp.pad(Jh, ((0, 0), (0, 0), (0, 0), (6, 6)))
    J = Jp[:, :, :, 0:_H]
    for t in range(1, _K):
        J = J + Jp[:, :, :, t:t + _H]

    accs = []
    for di in range(_DO):
        acc = J[di, 0][None, None] * ys[:, 0:_K]
        for e0 in range(1, _DO):
            acc = acc + J[di, e0][None, None] * ys[:, e0:e0 + _K]
        accs.append(acc)
    slabs = []
    for dv in range(_D):
        terms = [accs[di][:, dv - di]
                 for di in range(max(0, dv - _K + 1), min(_DO - 1, dv) + 1)]
        s = terms[0]
        for t in terms[1:]:
            s = s + t
        slabs.append(s[:, None])
    folded = jnp.concatenate(slabs, axis=1)

    dvi = lax.broadcasted_iota(jnp.int32, (_D, _H, _H), 0).astype(jnp.float32)
    hvi = lax.broadcasted_iota(jnp.int32, (_D, _H, _H), 1).astype(jnp.float32)
    wvi = lax.broadcasted_iota(jnp.int32, (_D, _H, _H), 2).astype(jnp.float32)
    covD = jnp.minimum(jnp.minimum(dvi + 1.0, float(_K)), float(_D) - dvi)
    covH = (jnp.minimum(hvi, float(_HO - 1))
            - jnp.maximum(hvi - float(_K - 1), 0.0) + 1.0)
    covW = (jnp.minimum(wvi, float(_HO - 1))
            - jnp.maximum(wvi - float(_K - 1), 0.0) + 1.0)
    wgt = covD * covH * covW

    total = jnp.sum(jnp.abs(xs * wgt[None] - folded))
    out_ref[0, 0] = total / float(_C * _D * _H * _H)


def kernel(x, y):
    xs = x.reshape(_C, _D, _H, _H)
    ys = y.reshape(_C, _D, _H, _H)
    out = pl.pallas_call(
        _loss_kernel,
        out_shape=jax.ShapeDtypeStruct((1, 1), jnp.float32),
        out_specs=pl.BlockSpec(memory_space=pltpu.SMEM),
    )(xs, ys)
    return out.reshape(())
